# baseline (speedup 1.0000x reference)
import functools

import jax
import jax.numpy as jnp
from jax import lax
from jax.experimental import pallas as pl
from jax.experimental.pallas import tpu as pltpu
from jax.experimental.pallas import tpu_sc as plsc

VOCAB = 1_000_000
FACTOR = 1000
CHUNK = 1000
ROWS = 64
NW = 32
RPW = ROWS // NW
BLK_CH = 16
BLK = BLK_CH * CHUNK
NBLK = RPW * FACTOR // BLK_CH
L = 16
FULL = CHUNK // L
NEG = -3.0e38

_mesh = plsc.VectorSubcoreMesh(core_axis_name="c", subcore_axis_name="s")


def _sum62(buf, start):
    accs = [jnp.zeros((L,), jnp.float32) for _ in range(4)]
    for j in range(FULL):
        v = jnp.exp(buf[pl.ds(start + j * L, L)])
        accs[j % 4] = accs[j % 4] + v
    return (accs[0] + accs[1]) + (accs[2] + accs[3])


@functools.partial(
    pl.kernel,
    out_type=jax.ShapeDtypeStruct((NW, L), jnp.int32),
    mesh=_mesh,
    compiler_params=pltpu.CompilerParams(needs_layout_passes=False),
    scratch_types=[
        pltpu.VMEM((BLK,), jnp.float32),
        pltpu.VMEM((BLK,), jnp.float32),
        pltpu.VMEM((RPW * FACTOR + L,), jnp.float32),
        pltpu.VMEM((RPW * FACTOR + L,), jnp.float32),
        pltpu.VMEM((RPW * CHUNK + L,), jnp.float32),
        pltpu.VMEM((CHUNK + L,), jnp.float32),
        pltpu.VMEM((L,), jnp.int32),
        pltpu.SemaphoreType.DMA,
        pltpu.SemaphoreType.DMA,
        pltpu.SemaphoreType.DMA,
        pltpu.SemaphoreType.DMA,
    ],
)
def _sampler(logits_hbm, e1_hbm, g2_hbm, out_hbm,
             buf0, buf1, s_v, e1_v, g2_v, ch_v, out_v,
             sem0, sem1, semg, semc):
    wid = lax.axis_index("s") * 2 + lax.axis_index("c")
    row0 = wid * RPW
    base = row0 * VOCAB
    iota = lax.iota(jnp.int32, L)
    lo8 = iota < 8

    pltpu.async_copy(e1_hbm.at[pl.ds(row0 * FACTOR, RPW * FACTOR)],
                     e1_v.at[pl.ds(0, RPW * FACTOR)], semg)
    pltpu.async_copy(g2_hbm.at[pl.ds(row0 * CHUNK, RPW * CHUNK)],
                     g2_v.at[pl.ds(0, RPW * CHUNK)], semg)

    def start_blk(q, buf, sem):
        pltpu.async_copy(logits_hbm.at[pl.ds(base + q * BLK, BLK)], buf, sem)

    def wait_blk(buf, sem):
        pltpu.make_async_copy(logits_hbm.at[pl.ds(0, BLK)], buf, sem).wait()

    def process(buf, q):
        def pair_body(p, sums):
            cbase = p * (2 * CHUNK)
            acc_a = _sum62(buf, cbase)
            mid = jnp.exp(buf[pl.ds(cbase + FULL * L, L)])
            s_a = jnp.sum(acc_a + jnp.where(lo8, mid, 0.0))
            acc_b = _sum62(buf, cbase + CHUNK + 8)
            s_b = jnp.sum(acc_b + jnp.where(lo8, 0.0, mid))
            sums = jnp.where(iota == 2 * p, s_a, sums)
            sums = jnp.where(iota == 2 * p + 1, s_b, sums)
            return sums
        sums = lax.fori_loop(0, BLK_CH // 2, pair_body,
                             jnp.zeros((L,), jnp.float32))
        s_v[pl.ds(q * BLK_CH, L)] = sums

    start_blk(0, buf0, sem0)
    def stream_body(g, _):
        q0 = 2 * g
        q1 = 2 * g + 1

        @pl.when(q1 < NBLK)
        def _():
            start_blk(q1, buf1, sem1)
        wait_blk(buf0, sem0)
        process(buf0, q0)

        @pl.when(q0 + 2 < NBLK)
        def _():
            start_blk(q0 + 2, buf0, sem0)

        @pl.when(q1 < NBLK)
        def _():
            wait_blk(buf1, sem1)
            process(buf1, q1)
        return 0
    lax.fori_loop(0, (NBLK + 1) // 2, stream_body, 0)

    pltpu.make_async_copy(e1_hbm.at[pl.ds(0, RPW * FACTOR)],
                          e1_v.at[pl.ds(0, RPW * FACTOR)], semg).wait()
    pltpu.make_async_copy(g2_hbm.at[pl.ds(0, RPW * CHUNK)],
                          g2_v.at[pl.ds(0, RPW * CHUNK)], semg).wait()

    def argmax1000(load_fn):
        def body(j, carry):
            best, bidx = carry
            u = load_fn(j)
            idxv = j * L + iota
            better = u > best
            return (jnp.where(better, u, best),
                    jnp.where(better, idxv, bidx))
        best, bidx = lax.fori_loop(
            0, FULL, body,
            (jnp.full((L,), NEG, jnp.float32), jnp.zeros((L,), jnp.int32)))
        u_t = load_fn(FULL)
        u_t = jnp.where(lo8, u_t, NEG)
        better = u_t > best
        best = jnp.where(better, u_t, best)
        bidx = jnp.where(better, FULL * L + iota, bidx)
        m = jnp.max(best)
        cand = jnp.where(best == m, bidx, 2 ** 30)
        return jnp.min(cand)

    out_vec = jnp.zeros((L,), jnp.int32)
    for k in range(RPW):
        def outer_load(j, _k=k):
            s = s_v[pl.ds(_k * FACTOR + j * L, L)]
            e = e1_v[pl.ds(_k * FACTOR + j * L, L)]
            u = s * e
            return jnp.where((j * L + iota) < FACTOR, u, NEG)
        f = argmax1000(outer_load)

        pltpu.async_copy(
            logits_hbm.at[pl.ds(base + k * VOCAB + f * CHUNK, CHUNK)],
            ch_v.at[pl.ds(0, CHUNK)], semc).wait()

        def inner_load(j, _k=k):
            x = ch_v[pl.ds(j * L, L)]
            g = g2_v[pl.ds(_k * CHUNK + j * L, L)]
            return jnp.where((j * L + iota) < CHUNK, x + g, NEG)
        c = argmax1000(inner_load)

        out_vec = jnp.where(iota == k, f * CHUNK + c, out_vec)

    out_v[...] = out_vec
    pltpu.sync_copy(out_v, out_hbm.at[wid])


def kernel(logits):
    b, s, v = logits.shape
    assert v == VOCAB and b * s == ROWS
    k1, k2 = jax.random.split(jax.random.key(42))
    e1 = jnp.exp(jax.random.gumbel(k1, (ROWS * FACTOR,), jnp.float32))
    g2 = jax.random.gumbel(k2, (ROWS * CHUNK,), jnp.float32)
    out = _sampler(logits.reshape(-1), e1, g2)
    return out[:, :RPW].reshape(b, s)


# speedup vs baseline: 1.0009x; 1.0009x over previous
import functools

import jax
import jax.numpy as jnp
from jax import lax
from jax.experimental import pallas as pl
from jax.experimental.pallas import tpu as pltpu
from jax.experimental.pallas import tpu_sc as plsc

VOCAB = 1_000_000
FACTOR = 1000
CHUNK = 1000
ROWS = 64
NW = 32
RPW = ROWS // NW
BLK_CH = 16
BLK = BLK_CH * CHUNK
NBLK = RPW * FACTOR // BLK_CH
L = 16
FULL = CHUNK // L
NEG = -3.0e38

_mesh = plsc.VectorSubcoreMesh(core_axis_name="c", subcore_axis_name="s")


def _pair_sums(buf, cbase, lo8):
    z = jnp.zeros((L,), jnp.float32)

    @plsc.parallel_loop(0, FULL // 2, carry=(z, z, z, z), unroll=2)
    def red(j, accs):
        a0, a1, b0, b1 = accs
        oa = cbase + j * (2 * L)
        ob = cbase + CHUNK + 8 + j * (2 * L)
        a0 = a0 + jnp.exp(buf[pl.ds(oa, L)])
        a1 = a1 + jnp.exp(buf[pl.ds(oa + L, L)])
        b0 = b0 + jnp.exp(buf[pl.ds(ob, L)])
        b1 = b1 + jnp.exp(buf[pl.ds(ob + L, L)])
        return (a0, a1, b0, b1)

    a0, a1, b0, b1 = red
    mid = jnp.exp(buf[pl.ds(cbase + FULL * L, L)])
    s_a = jnp.sum((a0 + a1) + jnp.where(lo8, mid, 0.0))
    s_b = jnp.sum((b0 + b1) + jnp.where(lo8, 0.0, mid))
    return s_a, s_b


@functools.partial(
    pl.kernel,
    out_type=jax.ShapeDtypeStruct((NW, L), jnp.int32),
    mesh=_mesh,
    compiler_params=pltpu.CompilerParams(needs_layout_passes=False),
    scratch_types=[
        pltpu.VMEM((BLK,), jnp.float32),
        pltpu.VMEM((BLK,), jnp.float32),
        pltpu.VMEM((RPW * FACTOR + L,), jnp.float32),
        pltpu.VMEM((RPW * FACTOR + L,), jnp.float32),
        pltpu.VMEM((RPW * CHUNK + L,), jnp.float32),
        pltpu.VMEM((CHUNK + L,), jnp.float32),
        pltpu.VMEM((L,), jnp.int32),
        pltpu.SemaphoreType.DMA,
        pltpu.SemaphoreType.DMA,
        pltpu.SemaphoreType.DMA,
        pltpu.SemaphoreType.DMA,
    ],
)
def _sampler(logits_hbm, e1_hbm, g2_hbm, out_hbm,
             buf0, buf1, s_v, e1_v, g2_v, ch_v, out_v,
             sem0, sem1, semg, semc):
    wid = lax.axis_index("s") * 2 + lax.axis_index("c")
    row0 = wid * RPW
    base = row0 * VOCAB
    iota = lax.iota(jnp.int32, L)
    lo8 = iota < 8

    pltpu.async_copy(e1_hbm.at[pl.ds(row0 * FACTOR, RPW * FACTOR)],
                     e1_v.at[pl.ds(0, RPW * FACTOR)], semg)
    pltpu.async_copy(g2_hbm.at[pl.ds(row0 * CHUNK, RPW * CHUNK)],
                     g2_v.at[pl.ds(0, RPW * CHUNK)], semg)

    def start_blk(q, buf, sem):
        pltpu.async_copy(logits_hbm.at[pl.ds(base + q * BLK, BLK)], buf, sem)

    def wait_blk(buf, sem):
        pltpu.make_async_copy(logits_hbm.at[pl.ds(0, BLK)], buf, sem).wait()

    def process(buf, q):
        def pair_body(p, sums):
            s_a, s_b = _pair_sums(buf, p * (2 * CHUNK), lo8)
            sums = jnp.where(iota == 2 * p, s_a, sums)
            sums = jnp.where(iota == 2 * p + 1, s_b, sums)
            return sums
        sums = lax.fori_loop(0, BLK_CH // 2, pair_body,
                             jnp.zeros((L,), jnp.float32))
        s_v[pl.ds(q * BLK_CH, L)] = sums

    start_blk(0, buf0, sem0)
    def stream_body(g, _):
        q0 = 2 * g
        q1 = 2 * g + 1

        @pl.when(q1 < NBLK)
        def _():
            start_blk(q1, buf1, sem1)
        wait_blk(buf0, sem0)
        process(buf0, q0)

        @pl.when(q0 + 2 < NBLK)
        def _():
            start_blk(q0 + 2, buf0, sem0)

        @pl.when(q1 < NBLK)
        def _():
            wait_blk(buf1, sem1)
            process(buf1, q1)
        return 0
    lax.fori_loop(0, (NBLK + 1) // 2, stream_body, 0)

    pltpu.make_async_copy(e1_hbm.at[pl.ds(0, RPW * FACTOR)],
                          e1_v.at[pl.ds(0, RPW * FACTOR)], semg).wait()
    pltpu.make_async_copy(g2_hbm.at[pl.ds(0, RPW * CHUNK)],
                          g2_v.at[pl.ds(0, RPW * CHUNK)], semg).wait()

    def argmax1000(load_fn):
        def body(j, carry):
            best, bidx = carry
            u = load_fn(j)
            idxv = j * L + iota
            better = u > best
            return (jnp.where(better, u, best),
                    jnp.where(better, idxv, bidx))
        best, bidx = lax.fori_loop(
            0, FULL, body,
            (jnp.full((L,), NEG, jnp.float32), jnp.zeros((L,), jnp.int32)))
        u_t = load_fn(FULL)
        u_t = jnp.where(lo8, u_t, NEG)
        better = u_t > best
        best = jnp.where(better, u_t, best)
        bidx = jnp.where(better, FULL * L + iota, bidx)
        m = jnp.max(best)
        cand = jnp.where(best == m, bidx, 2 ** 30)
        return jnp.min(cand)

    out_vec = jnp.zeros((L,), jnp.int32)
    for k in range(RPW):
        def outer_load(j, _k=k):
            s = s_v[pl.ds(_k * FACTOR + j * L, L)]
            e = e1_v[pl.ds(_k * FACTOR + j * L, L)]
            u = s * e
            return jnp.where((j * L + iota) < FACTOR, u, NEG)
        f = argmax1000(outer_load)

        pltpu.async_copy(
            logits_hbm.at[pl.ds(base + k * VOCAB + f * CHUNK, CHUNK)],
            ch_v.at[pl.ds(0, CHUNK)], semc).wait()

        def inner_load(j, _k=k):
            x = ch_v[pl.ds(j * L, L)]
            g = g2_v[pl.ds(_k * CHUNK + j * L, L)]
            return jnp.where((j * L + iota) < CHUNK, x + g, NEG)
        c = argmax1000(inner_load)

        out_vec = jnp.where(iota == k, f * CHUNK + c, out_vec)

    out_v[...] = out_vec
    pltpu.sync_copy(out_v, out_hbm.at[wid])


def kernel(logits):
    b, s, v = logits.shape
    assert v == VOCAB and b * s == ROWS
    k1, k2 = jax.random.split(jax.random.key(42))
    e1 = jnp.exp(jax.random.gumbel(k1, (ROWS * FACTOR,), jnp.float32))
    g2 = jax.random.gumbel(k2, (ROWS * CHUNK,), jnp.float32)
    out = _sampler(logits.reshape(-1), e1, g2)
    return out[:, :RPW].reshape(b, s)


# speedup vs baseline: 1.0036x; 1.0027x over previous
import functools

import jax
import jax.numpy as jnp
import numpy as np
from jax import lax
from jax.experimental import pallas as pl
from jax.experimental.pallas import tpu as pltpu
from jax.experimental.pallas import tpu_sc as plsc

VOCAB = 1_000_000
FACTOR = 1000
CHUNK = 1000
ROWS = 64
NW = 32
RPW = ROWS // NW
BLK_CH = 16
BLK = BLK_CH * CHUNK
NBLK = RPW * FACTOR // BLK_CH
L = 16
FULL = CHUNK // L
NEG = -3.0e38

_mesh = plsc.VectorSubcoreMesh(core_axis_name="c", subcore_axis_name="s")

_KEY1 = (np.uint32(1832780943), np.uint32(270669613))
_KEY2 = (np.uint32(64467757), np.uint32(2916123636))
_ROT = ((13, 15, 26, 6), (17, 29, 16, 24))


def _np_uniform(key, n):
    x0 = np.zeros(n, np.uint32)
    x1 = np.arange(n, dtype=np.uint32)
    k0, k1 = key
    ks = (k0, k1, np.uint32(k0 ^ k1 ^ np.uint32(0x1BD11BDA)))
    x0 += ks[0]
    x1 += ks[1]
    for i in range(5):
        for r in _ROT[i % 2]:
            x0 += x1
            x1 = ((x1 << np.uint32(r)) | (x1 >> np.uint32(32 - r))).astype(np.uint32)
            x1 ^= x0
        x0 += ks[(i + 1) % 3]
        x1 += ks[(i + 2) % 3] + np.uint32(i + 1)
    bits = x0 ^ x1
    f = ((bits >> np.uint32(9)) | np.uint32(0x3F800000)).view(np.float32) \
        - np.float32(1.0)
    tiny = np.float32(np.finfo(np.float32).tiny)
    return np.maximum(tiny, f * (np.float32(1.0) - tiny) + tiny)


@functools.lru_cache(maxsize=None)
def _gumbel_uniforms():
    return (_np_uniform(_KEY1, ROWS * FACTOR), _np_uniform(_KEY2, ROWS * CHUNK))


def _pair_sums(buf, cbase, lo8):
    z = jnp.zeros((L,), jnp.float32)

    @plsc.parallel_loop(0, FULL // 2, carry=(z, z, z, z), unroll=2)
    def red(j, accs):
        a0, a1, b0, b1 = accs
        oa = cbase + j * (2 * L)
        ob = cbase + CHUNK + 8 + j * (2 * L)
        a0 = a0 + jnp.exp(buf[pl.ds(oa, L)])
        a1 = a1 + jnp.exp(buf[pl.ds(oa + L, L)])
        b0 = b0 + jnp.exp(buf[pl.ds(ob, L)])
        b1 = b1 + jnp.exp(buf[pl.ds(ob + L, L)])
        return (a0, a1, b0, b1)

    a0, a1, b0, b1 = red
    mid = jnp.exp(buf[pl.ds(cbase + FULL * L, L)])
    s_a = jnp.sum((a0 + a1) + jnp.where(lo8, mid, 0.0))
    s_b = jnp.sum((b0 + b1) + jnp.where(lo8, 0.0, mid))
    return s_a, s_b


@functools.partial(
    pl.kernel,
    out_type=jax.ShapeDtypeStruct((NW, L), jnp.int32),
    mesh=_mesh,
    compiler_params=pltpu.CompilerParams(needs_layout_passes=False),
    scratch_types=[
        pltpu.VMEM((BLK,), jnp.float32),
        pltpu.VMEM((BLK,), jnp.float32),
        pltpu.VMEM((RPW * FACTOR + L,), jnp.float32),
        pltpu.VMEM((RPW * FACTOR + L,), jnp.float32),
        pltpu.VMEM((RPW * CHUNK + L,), jnp.float32),
        pltpu.VMEM((CHUNK + L,), jnp.float32),
        pltpu.VMEM((L,), jnp.int32),
        pltpu.SemaphoreType.DMA,
        pltpu.SemaphoreType.DMA,
        pltpu.SemaphoreType.DMA,
        pltpu.SemaphoreType.DMA,
    ],
)
def _sampler(logits_hbm, e1_hbm, g2_hbm, out_hbm,
             buf0, buf1, s_v, e1_v, g2_v, ch_v, out_v,
             sem0, sem1, semg, semc):
    wid = lax.axis_index("s") * 2 + lax.axis_index("c")
    row0 = wid * RPW
    base = row0 * VOCAB
    iota = lax.iota(jnp.int32, L)
    lo8 = iota < 8

    pltpu.async_copy(e1_hbm.at[pl.ds(row0 * FACTOR, RPW * FACTOR)],
                     e1_v.at[pl.ds(0, RPW * FACTOR)], semg)
    pltpu.async_copy(g2_hbm.at[pl.ds(row0 * CHUNK, RPW * CHUNK)],
                     g2_v.at[pl.ds(0, RPW * CHUNK)], semg)

    def start_blk(q, buf, sem):
        pltpu.async_copy(logits_hbm.at[pl.ds(base + q * BLK, BLK)], buf, sem)

    def wait_blk(buf, sem):
        pltpu.make_async_copy(logits_hbm.at[pl.ds(0, BLK)], buf, sem).wait()

    def process(buf, q):
        def pair_body(p, sums):
            s_a, s_b = _pair_sums(buf, p * (2 * CHUNK), lo8)
            sums = jnp.where(iota == 2 * p, s_a, sums)
            sums = jnp.where(iota == 2 * p + 1, s_b, sums)
            return sums
        sums = lax.fori_loop(0, BLK_CH // 2, pair_body,
                             jnp.zeros((L,), jnp.float32))
        s_v[pl.ds(q * BLK_CH, L)] = sums

    start_blk(0, buf0, sem0)
    def stream_body(g, _):
        q0 = 2 * g
        q1 = 2 * g + 1

        @pl.when(q1 < NBLK)
        def _():
            start_blk(q1, buf1, sem1)
        wait_blk(buf0, sem0)
        process(buf0, q0)

        @pl.when(q0 + 2 < NBLK)
        def _():
            start_blk(q0 + 2, buf0, sem0)

        @pl.when(q1 < NBLK)
        def _():
            wait_blk(buf1, sem1)
            process(buf1, q1)
        return 0
    lax.fori_loop(0, (NBLK + 1) // 2, stream_body, 0)

    pltpu.make_async_copy(e1_hbm.at[pl.ds(0, RPW * FACTOR)],
                          e1_v.at[pl.ds(0, RPW * FACTOR)], semg).wait()
    pltpu.make_async_copy(g2_hbm.at[pl.ds(0, RPW * CHUNK)],
                          g2_v.at[pl.ds(0, RPW * CHUNK)], semg).wait()

    def argmax1000(load_fn):
        def body(j, carry):
            best, bidx = carry
            u = load_fn(j)
            idxv = j * L + iota
            better = u > best
            return (jnp.where(better, u, best),
                    jnp.where(better, idxv, bidx))
        best, bidx = lax.fori_loop(
            0, FULL, body,
            (jnp.full((L,), NEG, jnp.float32), jnp.zeros((L,), jnp.int32)))
        u_t = load_fn(FULL)
        u_t = jnp.where(lo8, u_t, NEG)
        better = u_t > best
        best = jnp.where(better, u_t, best)
        bidx = jnp.where(better, FULL * L + iota, bidx)
        m = jnp.max(best)
        cand = jnp.where(best == m, bidx, 2 ** 30)
        return jnp.min(cand)

    out_vec = jnp.zeros((L,), jnp.int32)
    for k in range(RPW):
        def outer_load(j, _k=k):
            s = s_v[pl.ds(_k * FACTOR + j * L, L)]
            e = e1_v[pl.ds(_k * FACTOR + j * L, L)]
            u = s * e
            return jnp.where((j * L + iota) < FACTOR, u, NEG)
        f = argmax1000(outer_load)

        pltpu.async_copy(
            logits_hbm.at[pl.ds(base + k * VOCAB + f * CHUNK, CHUNK)],
            ch_v.at[pl.ds(0, CHUNK)], semc).wait()

        def inner_load(j, _k=k):
            x = ch_v[pl.ds(j * L, L)]
            g = g2_v[pl.ds(_k * CHUNK + j * L, L)]
            return jnp.where((j * L + iota) < CHUNK, x + g, NEG)
        c = argmax1000(inner_load)

        out_vec = jnp.where(iota == k, f * CHUNK + c, out_vec)

    out_v[...] = out_vec
    pltpu.sync_copy(out_v, out_hbm.at[wid])


def kernel(logits):
    b, s, v = logits.shape
    assert v == VOCAB and b * s == ROWS
    u1, u2 = _gumbel_uniforms()
    e1 = jnp.exp(-jnp.log(-jnp.log(jnp.asarray(u1))))
    g2 = -jnp.log(-jnp.log(jnp.asarray(u2)))
    out = _sampler(logits.reshape(-1), e1, g2)
    return out[:, :RPW].reshape(b, s)


# speedup vs baseline: 27.8209x; 27.7200x over previous
import functools

import jax
import jax.numpy as jnp
import numpy as np
from jax import lax
from jax.experimental import pallas as pl
from jax.experimental.pallas import tpu as pltpu
from jax.experimental.pallas import tpu_sc as plsc

VOCAB = 1_000_000
FACTOR = 1000
CHUNK = 1000
ROWS = 64
NW = 32
RPW = ROWS // NW
BLK_CH = 8
BLK = BLK_CH * CHUNK
SPAN = 8320
CAP_O = 991744
NBLK = 63
HALF_CH = 496
L = 16
FULL = CHUNK // L
NEG = -3.0e38

_mesh = plsc.VectorSubcoreMesh(core_axis_name="c", subcore_axis_name="s")

_KEY1 = (np.uint32(1832780943), np.uint32(270669613))
_KEY2 = (np.uint32(64467757), np.uint32(2916123636))
_ROT = ((13, 15, 26, 6), (17, 29, 16, 24))


def _np_uniform(key, n):
    x0 = np.zeros(n, np.uint32)
    x1 = np.arange(n, dtype=np.uint32)
    k0, k1 = key
    ks = (k0, k1, np.uint32(k0 ^ k1 ^ np.uint32(0x1BD11BDA)))
    x0 += ks[0]
    x1 += ks[1]
    for i in range(5):
        for r in _ROT[i % 2]:
            x0 += x1
            x1 = ((x1 << np.uint32(r)) | (x1 >> np.uint32(32 - r))).astype(np.uint32)
            x1 ^= x0
        x0 += ks[(i + 1) % 3]
        x1 += ks[(i + 2) % 3] + np.uint32(i + 1)
    bits = x0 ^ x1
    f = ((bits >> np.uint32(9)) | np.uint32(0x3F800000)).view(np.float32) \
        - np.float32(1.0)
    tiny = np.float32(np.finfo(np.float32).tiny)
    return np.maximum(tiny, f * (np.float32(1.0) - tiny) + tiny)


@functools.lru_cache(maxsize=None)
def _gumbel_uniforms():
    return (_np_uniform(_KEY1, ROWS * FACTOR), _np_uniform(_KEY2, ROWS * CHUNK))


def _pair_sums(buf, d1, cbase, lo8):
    z = jnp.zeros((L,), jnp.float32)

    @plsc.parallel_loop(0, FULL // 2, carry=(z, z, z, z), unroll=2)
    def red(j, accs):
        a0, a1, b0, b1 = accs
        oa = cbase + j * (2 * L)
        ob = cbase + CHUNK + 8 + j * (2 * L)
        a0 = a0 + jnp.exp(buf[d1, pl.ds(oa, L)])
        a1 = a1 + jnp.exp(buf[d1, pl.ds(oa + L, L)])
        b0 = b0 + jnp.exp(buf[d1, pl.ds(ob, L)])
        b1 = b1 + jnp.exp(buf[d1, pl.ds(ob + L, L)])
        return (a0, a1, b0, b1)

    a0, a1, b0, b1 = red
    mid = jnp.exp(buf[d1, pl.ds(cbase + FULL * L, L)])
    s_a = jnp.sum((a0 + a1) + jnp.where(lo8, mid, 0.0))
    s_b = jnp.sum((b0 + b1) + jnp.where(lo8, 0.0, mid))
    return s_a, s_b


@functools.partial(
    pl.kernel,
    out_type=jax.ShapeDtypeStruct((NW, L), jnp.int32),
    mesh=_mesh,
    compiler_params=pltpu.CompilerParams(needs_layout_passes=False),
    scratch_types=[
        pltpu.VMEM((4, SPAN), jnp.float32),
        pltpu.VMEM((4, SPAN), jnp.float32),
        pltpu.VMEM((4 * 512,), jnp.float32),
        pltpu.VMEM_SHARED((32 * 1008,), jnp.float32),
        pltpu.VMEM((1008,), jnp.float32),
        pltpu.VMEM((RPW * FACTOR + L,), jnp.float32),
        pltpu.VMEM((RPW * CHUNK + L,), jnp.float32),
        pltpu.VMEM((4, 1152), jnp.float32),
        pltpu.VMEM((L,), jnp.int32),
        pltpu.SemaphoreType.DMA,
        pltpu.SemaphoreType.DMA,
        pltpu.SemaphoreType.DMA,
        pltpu.SemaphoreType.DMA,
    ],
)
def _sampler(logits_hbm, e1_hbm, g2_hbm, out_hbm,
             buf0, buf1, s_v, s_sh, s2_v, e1_v, g2_v, ch_v, out_v,
             sem0, sem1, semg, semc):
    cid = lax.axis_index("c")
    sid = lax.axis_index("s")
    wid = cid * 16 + sid
    d0 = wid // 2
    h = wid % 2
    qs = 62 * h
    iota = lax.iota(jnp.int32, L)
    lo8 = iota < 8

    row0 = wid * RPW
    pltpu.async_copy(e1_hbm.at[pl.ds(row0 * FACTOR, RPW * FACTOR)],
                     e1_v.at[pl.ds(0, RPW * FACTOR)], semg)
    pltpu.async_copy(g2_hbm.at[pl.ds(row0 * CHUNK, RPW * CHUNK)],
                     g2_v.at[pl.ds(0, RPW * CHUNK)], semg)

    def blk_off(q):
        return jnp.minimum((q * BLK) // 128 * 128, CAP_O)

    def start_blk(q, buf, sem):
        pltpu.async_copy(
            logits_hbm.at[d0, pl.ds(0, 4), pl.ds(blk_off(q), SPAN)], buf, sem)

    def wait_blk(buf, sem):
        pltpu.make_async_copy(
            logits_hbm.at[0, pl.ds(0, 4), pl.ds(0, SPAN)], buf, sem).wait()

    def process(buf, q, half, sums4):
        delta = q * BLK - blk_off(q)
        new = []
        for d1 in range(4):

            def pair_body(p, sums, _d1=d1, _delta=delta):
                s_a, s_b = _pair_sums(buf, _d1, _delta + p * (2 * CHUNK), lo8)
                sums = jnp.where(iota == half * 8 + 2 * p, s_a, sums)
                return jnp.where(iota == half * 8 + 2 * p + 1, s_b, sums)

            new.append(lax.fori_loop(0, BLK_CH // 2, pair_body, sums4[d1]))
        return tuple(new)

    z = jnp.zeros((L,), jnp.float32)
    start_blk(qs, buf0, sem0)

    def stream_body(g, sums4):
        q0 = qs + 2 * g
        start_blk(q0 + 1, buf1, sem1)
        wait_blk(buf0, sem0)
        sums4 = process(buf0, q0, 0, sums4)
        start_blk(q0 + 2, buf0, sem0)
        wait_blk(buf1, sem1)
        sums4 = process(buf1, q0 + 1, 1, sums4)
        for d1 in range(4):
            s_v[pl.ds(d1 * 512 + g * L, L)] = sums4[d1]
        return sums4

    sums4 = lax.fori_loop(0, (NBLK - 1) // 2, stream_body, (z, z, z, z))
    wait_blk(buf0, sem0)
    sums4 = process(buf0, qs + 62, 0, sums4)
    for d1 in range(4):
        s_v[pl.ds(d1 * 512 + HALF_CH, L)] = sums4[d1]

    for d1 in range(4):
        r_loc = 4 * (sid // 2) + d1

        @pl.when(h == 0)
        def _(_d1=d1, _r=r_loc):
            pltpu.sync_copy(s_v.at[pl.ds(_d1 * 512, HALF_CH)],
                            s_sh.at[pl.ds(_r * 1008, HALF_CH)])

        @pl.when(h == 1)
        def _(_d1=d1, _r=r_loc):
            pltpu.sync_copy(s_v.at[pl.ds(_d1 * 512, 512)],
                            s_sh.at[pl.ds(_r * 1008 + HALF_CH, 512)])

    plsc.subcore_barrier()

    pltpu.make_async_copy(e1_hbm.at[pl.ds(0, RPW * FACTOR)],
                          e1_v.at[pl.ds(0, RPW * FACTOR)], semg).wait()
    pltpu.make_async_copy(g2_hbm.at[pl.ds(0, RPW * CHUNK)],
                          g2_v.at[pl.ds(0, RPW * CHUNK)], semg).wait()

    def argmax1000(load_fn):
        def body(j, carry):
            best, bidx = carry
            u = load_fn(j)
            idxv = j * L + iota
            better = u > best
            return (jnp.where(better, u, best),
                    jnp.where(better, idxv, bidx))
        best, bidx = lax.fori_loop(
            0, FULL, body,
            (jnp.full((L,), NEG, jnp.float32), jnp.zeros((L,), jnp.int32)))
        u_t = load_fn(FULL)
        u_t = jnp.where(lo8, u_t, NEG)
        better = u_t > best
        best = jnp.where(better, u_t, best)
        bidx = jnp.where(better, FULL * L + iota, bidx)
        m = jnp.max(best)
        cand = jnp.where(best == m, bidx, 2 ** 30)
        return jnp.min(cand)

    out_vec = jnp.zeros((L,), jnp.int32)
    for k in range(RPW):
        r = row0 + k
        r_loc = 2 * sid + k
        pltpu.sync_copy(s_sh.at[pl.ds(r_loc * 1008, 1008)],
                        s2_v.at[pl.ds(0, 1008)])

        def outer_load(j, _k=k):
            s = s2_v[pl.ds(j * L, L)]
            e = e1_v[pl.ds(_k * FACTOR + j * L, L)]
            return jnp.where((j * L + iota) < FACTOR, s * e, NEG)
        f = argmax1000(outer_load)

        g_off = jnp.minimum((f * CHUNK) // 128 * 128, 998912)
        g_delta = f * CHUNK - g_off
        pltpu.async_copy(
            logits_hbm.at[r // 4, pl.ds(0, 4), pl.ds(g_off, 1152)],
            ch_v, semc).wait()
        d1g = r % 4

        def inner_load(j, _k=k, _d1g=d1g, _delta=g_delta):
            x = ch_v[_d1g, pl.ds(_delta + j * L, L)]
            g = g2_v[pl.ds(_k * CHUNK + j * L, L)]
            return jnp.where((j * L + iota) < CHUNK, x + g, NEG)
        c = argmax1000(inner_load)

        out_vec = jnp.where(iota == k, f * CHUNK + c, out_vec)

    out_v[...] = out_vec
    pltpu.sync_copy(out_v, out_hbm.at[wid])


def kernel(logits):
    b, s, v = logits.shape
    assert v == VOCAB and b * s == ROWS
    u1, u2 = _gumbel_uniforms()
    e1 = jnp.exp(-jnp.log(-jnp.log(jnp.asarray(u1))))
    g2 = -jnp.log(-jnp.log(jnp.asarray(u2)))
    out = _sampler(logits, e1, g2)
    return out[:, :RPW].reshape(b, s)


# speedup vs baseline: 30.8270x; 1.1081x over previous
import functools

import jax
import jax.numpy as jnp
import numpy as np
from jax import lax
from jax.experimental import pallas as pl
from jax.experimental.pallas import tpu as pltpu
from jax.experimental.pallas import tpu_sc as plsc

VOCAB = 1_000_000
FACTOR = 1000
CHUNK = 1000
ROWS = 64
NW = 32
RPW = ROWS // NW
BLK_CH = 8
BLK = BLK_CH * CHUNK
SPAN = 8320
CAP_O = 991744
NBLK = 63
HALF_CH = 496
L = 16
FULL = CHUNK // L
NEG = -3.0e38

_mesh = plsc.VectorSubcoreMesh(core_axis_name="c", subcore_axis_name="s")

_KEY1 = (np.uint32(1832780943), np.uint32(270669613))
_KEY2 = (np.uint32(64467757), np.uint32(2916123636))
_ROT = ((13, 15, 26, 6), (17, 29, 16, 24))


def _np_uniform(key, n):
    x0 = np.zeros(n, np.uint32)
    x1 = np.arange(n, dtype=np.uint32)
    k0, k1 = key
    ks = (k0, k1, np.uint32(k0 ^ k1 ^ np.uint32(0x1BD11BDA)))
    x0 += ks[0]
    x1 += ks[1]
    for i in range(5):
        for r in _ROT[i % 2]:
            x0 += x1
            x1 = ((x1 << np.uint32(r)) | (x1 >> np.uint32(32 - r))).astype(np.uint32)
            x1 ^= x0
        x0 += ks[(i + 1) % 3]
        x1 += ks[(i + 2) % 3] + np.uint32(i + 1)
    bits = x0 ^ x1
    f = ((bits >> np.uint32(9)) | np.uint32(0x3F800000)).view(np.float32) \
        - np.float32(1.0)
    tiny = np.float32(np.finfo(np.float32).tiny)
    return np.maximum(tiny, f * (np.float32(1.0) - tiny) + tiny)


@functools.lru_cache(maxsize=None)
def _gumbel_uniforms():
    return (_np_uniform(_KEY1, ROWS * FACTOR), _np_uniform(_KEY2, ROWS * CHUNK))


def _row_block_sums(buf, d1, delta, lo8, iota, half, sums):
    z = jnp.zeros((L,), jnp.float32)

    @plsc.parallel_loop(0, FULL, carry=(z,) * BLK_CH, unroll=2)
    def red(j, accs):
        o = delta + j * L
        return tuple(
            acc + jnp.exp(buf[d1, pl.ds(o + m * CHUNK, L)])
            for m, acc in enumerate(accs)
        )

    for m in range(BLK_CH):
        mid = jnp.exp(buf[d1, pl.ds(delta + m * CHUNK + FULL * L, L)])
        s_m = jnp.sum(red[m] + jnp.where(lo8, mid, 0.0))
        sums = jnp.where(iota == half * 8 + m, s_m, sums)
    return sums


@functools.partial(
    pl.kernel,
    out_type=jax.ShapeDtypeStruct((NW, L), jnp.int32),
    mesh=_mesh,
    compiler_params=pltpu.CompilerParams(needs_layout_passes=False),
    scratch_types=[
        pltpu.VMEM((4, SPAN), jnp.float32),
        pltpu.VMEM((4, SPAN), jnp.float32),
        pltpu.VMEM((4 * 512,), jnp.float32),
        pltpu.VMEM_SHARED((32 * 1008,), jnp.float32),
        pltpu.VMEM((1008,), jnp.float32),
        pltpu.VMEM((RPW * FACTOR + L,), jnp.float32),
        pltpu.VMEM((RPW * CHUNK + L,), jnp.float32),
        pltpu.VMEM((4, 1152), jnp.float32),
        pltpu.VMEM((L,), jnp.int32),
        pltpu.SemaphoreType.DMA,
        pltpu.SemaphoreType.DMA,
        pltpu.SemaphoreType.DMA,
        pltpu.SemaphoreType.DMA,
    ],
)
def _sampler(logits_hbm, e1_hbm, g2_hbm, out_hbm,
             buf0, buf1, s_v, s_sh, s2_v, e1_v, g2_v, ch_v, out_v,
             sem0, sem1, semg, semc):
    cid = lax.axis_index("c")
    sid = lax.axis_index("s")
    wid = cid * 16 + sid
    d0 = wid // 2
    h = wid % 2
    qs = 62 * h
    iota = lax.iota(jnp.int32, L)
    lo8 = iota < 8

    row0 = wid * RPW
    pltpu.async_copy(e1_hbm.at[pl.ds(row0 * FACTOR, RPW * FACTOR)],
                     e1_v.at[pl.ds(0, RPW * FACTOR)], semg)
    pltpu.async_copy(g2_hbm.at[pl.ds(row0 * CHUNK, RPW * CHUNK)],
                     g2_v.at[pl.ds(0, RPW * CHUNK)], semg)

    def blk_off(q):
        return jnp.minimum((q * BLK) // 128 * 128, CAP_O)

    def start_blk(q, buf, sem):
        pltpu.async_copy(
            logits_hbm.at[d0, pl.ds(0, 4), pl.ds(blk_off(q), SPAN)], buf, sem)

    def wait_blk(buf, sem):
        pltpu.make_async_copy(
            logits_hbm.at[0, pl.ds(0, 4), pl.ds(0, SPAN)], buf, sem).wait()

    def process(buf, q, half, sums4):
        delta = q * BLK - blk_off(q)
        return tuple(
            _row_block_sums(buf, d1, delta, lo8, iota, half, sums4[d1])
            for d1 in range(4)
        )

    z = jnp.zeros((L,), jnp.float32)
    start_blk(qs, buf0, sem0)

    def stream_body(g, sums4):
        q0 = qs + 2 * g
        start_blk(q0 + 1, buf1, sem1)
        wait_blk(buf0, sem0)
        sums4 = process(buf0, q0, 0, sums4)
        start_blk(q0 + 2, buf0, sem0)
        wait_blk(buf1, sem1)
        sums4 = process(buf1, q0 + 1, 1, sums4)
        for d1 in range(4):
            s_v[pl.ds(d1 * 512 + g * L, L)] = sums4[d1]
        return sums4

    sums4 = lax.fori_loop(0, (NBLK - 1) // 2, stream_body, (z, z, z, z))
    wait_blk(buf0, sem0)
    sums4 = process(buf0, qs + 62, 0, sums4)
    for d1 in range(4):
        s_v[pl.ds(d1 * 512 + HALF_CH, L)] = sums4[d1]

    for d1 in range(4):
        r_loc = 4 * (sid // 2) + d1

        @pl.when(h == 0)
        def _(_d1=d1, _r=r_loc):
            pltpu.sync_copy(s_v.at[pl.ds(_d1 * 512, HALF_CH)],
                            s_sh.at[pl.ds(_r * 1008, HALF_CH)])

        @pl.when(h == 1)
        def _(_d1=d1, _r=r_loc):
            pltpu.sync_copy(s_v.at[pl.ds(_d1 * 512, 512)],
                            s_sh.at[pl.ds(_r * 1008 + HALF_CH, 512)])

    plsc.subcore_barrier()

    pltpu.make_async_copy(e1_hbm.at[pl.ds(0, RPW * FACTOR)],
                          e1_v.at[pl.ds(0, RPW * FACTOR)], semg).wait()
    pltpu.make_async_copy(g2_hbm.at[pl.ds(0, RPW * CHUNK)],
                          g2_v.at[pl.ds(0, RPW * CHUNK)], semg).wait()

    def argmax1000(load_fn):
        def body(j, carry):
            best, bidx = carry
            u = load_fn(j)
            idxv = j * L + iota
            better = u > best
            return (jnp.where(better, u, best),
                    jnp.where(better, idxv, bidx))
        best, bidx = lax.fori_loop(
            0, FULL, body,
            (jnp.full((L,), NEG, jnp.float32), jnp.zeros((L,), jnp.int32)))
        u_t = load_fn(FULL)
        u_t = jnp.where(lo8, u_t, NEG)
        better = u_t > best
        best = jnp.where(better, u_t, best)
        bidx = jnp.where(better, FULL * L + iota, bidx)
        m = jnp.max(best)
        cand = jnp.where(best == m, bidx, 2 ** 30)
        return jnp.min(cand)

    out_vec = jnp.zeros((L,), jnp.int32)
    for k in range(RPW):
        r = row0 + k
        r_loc = 2 * sid + k
        pltpu.sync_copy(s_sh.at[pl.ds(r_loc * 1008, 1008)],
                        s2_v.at[pl.ds(0, 1008)])

        def outer_load(j, _k=k):
            s = s2_v[pl.ds(j * L, L)]
            e = e1_v[pl.ds(_k * FACTOR + j * L, L)]
            return jnp.where((j * L + iota) < FACTOR, s * e, NEG)
        f = argmax1000(outer_load)

        g_off = jnp.minimum((f * CHUNK) // 128 * 128, 998912)
        g_delta = f * CHUNK - g_off
        pltpu.async_copy(
            logits_hbm.at[r // 4, pl.ds(0, 4), pl.ds(g_off, 1152)],
            ch_v, semc).wait()
        d1g = r % 4

        def inner_load(j, _k=k, _d1g=d1g, _delta=g_delta):
            x = ch_v[_d1g, pl.ds(_delta + j * L, L)]
            g = g2_v[pl.ds(_k * CHUNK + j * L, L)]
            return jnp.where((j * L + iota) < CHUNK, x + g, NEG)
        c = argmax1000(inner_load)

        out_vec = jnp.where(iota == k, f * CHUNK + c, out_vec)

    out_v[...] = out_vec
    pltpu.sync_copy(out_v, out_hbm.at[wid])


def kernel(logits):
    b, s, v = logits.shape
    assert v == VOCAB and b * s == ROWS
    u1, u2 = _gumbel_uniforms()
    e1 = jnp.exp(-jnp.log(-jnp.log(jnp.asarray(u1))))
    g2 = -jnp.log(-jnp.log(jnp.asarray(u2)))
    out = _sampler(logits, e1, g2)
    return out[:, :RPW].reshape(b, s)
